# baseline (speedup 1.0000x reference)
import functools

import jax
import jax.numpy as jnp
from jax import lax
from jax.experimental import pallas as pl
from jax.experimental.pallas import tpu as pltpu
from jax.experimental.pallas import tpu_sc as plsc

_RING = 16
_LANE = 16


@functools.lru_cache(maxsize=None)
def _build(B, V, D):
    info = plsc.get_sparse_core_info()
    NC, NS = info.num_cores, info.num_subcores
    NW = NC * NS
    b_per_w = B // NW
    n_waves = b_per_w // _RING
    mesh = plsc.VectorSubcoreMesh(core_axis_name="c", subcore_axis_name="s")

    @functools.partial(
        pl.kernel,
        mesh=mesh,
        out_type=jax.ShapeDtypeStruct((D, B), jnp.float32),
        scratch_types=[
            pltpu.VMEM((b_per_w + _LANE,), jnp.int32),
            pltpu.VMEM((_RING, D, 128), jnp.float32),
            pltpu.VMEM((D, b_per_w), jnp.float32),
        ]
        + [pltpu.SemaphoreType.DMA] * _RING,
        compiler_params=pltpu.CompilerParams(needs_layout_passes=False),
    )
    def emb(idx_hbm, table_t_hbm, out_hbm, idx_v, blk_v, pan_v, *sems):
        wid = lax.axis_index("s") * NC + lax.axis_index("c")
        base = wid * b_per_w
        pltpu.sync_copy(idx_hbm.at[pl.ds(base, b_per_w)],
                        idx_v.at[pl.ds(0, b_per_w)])
        c16 = lax.iota(jnp.int32, _LANE)

        def fire(r, j):
            qb = pl.multiple_of(r - (r & 127), 128)
            pltpu.async_copy(
                table_t_hbm.at[:, pl.ds(qb, 128)], blk_v.at[j], sems[j]
            )

        def drain(j):
            pltpu.make_async_copy(
                table_t_hbm.at[:, pl.ds(0, 128)], blk_v.at[j], sems[j]
            ).wait()

        def extract(r, t, j):
            col = jnp.full((_LANE,), r & 127, jnp.int32)
            tcol = jnp.full((_LANE,), t, jnp.int32)
            for h in range(D // _LANE):
                v = plsc.load_gather(blk_v.at[j], [c16 + h * _LANE, col])
                plsc.store_scatter(pan_v, [c16 + h * _LANE, tcol], v)

        ids0 = idx_v[pl.ds(0, _LANE)]
        for j in range(_RING):
            fire(ids0[j], j)

        def wave(k, _):
            idsp = idx_v[pl.ds((k - 1) * _RING, _LANE)]
            idsn = idx_v[pl.ds(k * _RING, _LANE)]
            for j in range(_RING):
                drain(j)
                extract(idsp[j], (k - 1) * _RING + j, j)

                @pl.when(k < n_waves)
                def _():
                    fire(idsn[j], j)

            return ()

        lax.fori_loop(1, n_waves + 1, wave, (), unroll=False)
        pltpu.sync_copy(pan_v, out_hbm.at[:, pl.ds(base, b_per_w)])

    return emb


def kernel(token_id, embedding_table):
    B = token_id.shape[0]
    V, D = embedding_table.shape
    out_t = _build(B, V, D)(token_id, embedding_table.T)
    return out_t.T


# speedup vs baseline: 1.0349x; 1.0349x over previous
import functools

import jax
import jax.numpy as jnp
from jax import lax
from jax.experimental import pallas as pl
from jax.experimental.pallas import tpu as pltpu
from jax.experimental.pallas import tpu_sc as plsc

_RING = 8
_LANE = 16


@functools.lru_cache(maxsize=None)
def _build(B, V, D):
    info = plsc.get_sparse_core_info()
    NC, NS = info.num_cores, info.num_subcores
    NW = NC * NS
    b_per_w = B // NW
    n_waves = b_per_w // _RING
    mesh = plsc.VectorSubcoreMesh(core_axis_name="c", subcore_axis_name="s")

    @functools.partial(
        pl.kernel,
        mesh=mesh,
        out_type=jax.ShapeDtypeStruct((D, B), jnp.float32),
        scratch_types=[
            pltpu.VMEM((b_per_w + _LANE,), jnp.int32),
            pltpu.VMEM((_RING, D, 128), jnp.float32),
            pltpu.VMEM((D, b_per_w), jnp.float32),
        ]
        + [pltpu.SemaphoreType.DMA] * _RING,
        compiler_params=pltpu.CompilerParams(needs_layout_passes=False),
    )
    def emb(idx_hbm, table_t_hbm, out_hbm, idx_v, blk_v, pan_v, *sems):
        wid = lax.axis_index("s") * NC + lax.axis_index("c")
        base = wid * b_per_w
        pltpu.sync_copy(idx_hbm.at[pl.ds(base, b_per_w)],
                        idx_v.at[pl.ds(0, b_per_w)])
        c16 = lax.iota(jnp.int32, _LANE)

        def fire(r, j):
            qb = pl.multiple_of(r - (r & 127), 128)
            pltpu.async_copy(
                table_t_hbm.at[:, pl.ds(qb, 128)], blk_v.at[j], sems[j]
            )

        def drain(j):
            pltpu.make_async_copy(
                table_t_hbm.at[:, pl.ds(0, 128)], blk_v.at[j], sems[j]
            ).wait()

        def extract(r, t, j):
            col = jnp.full((_LANE,), r & 127, jnp.int32)
            tcol = jnp.full((_LANE,), t, jnp.int32)
            for h in range(D // _LANE):
                v = plsc.load_gather(blk_v.at[j], [c16 + h * _LANE, col])
                plsc.store_scatter(pan_v, [c16 + h * _LANE, tcol], v)

        ids0 = idx_v[pl.ds(0, _LANE)]
        for j in range(_RING):
            fire(ids0[j], j)

        def wave(k, _):
            idsp = idx_v[pl.ds((k - 1) * _RING, _LANE)]
            idsn = idx_v[pl.ds(k * _RING, _LANE)]
            for j in range(_RING):
                drain(j)
                extract(idsp[j], (k - 1) * _RING + j, j)

                @pl.when(k < n_waves)
                def _():
                    fire(idsn[j], j)

            return ()

        lax.fori_loop(1, n_waves + 1, wave, (), unroll=False)
        pltpu.sync_copy(pan_v, out_hbm.at[:, pl.ds(base, b_per_w)])

    return emb


def kernel(token_id, embedding_table):
    B = token_id.shape[0]
    V, D = embedding_table.shape
    out_t = _build(B, V, D)(token_id, embedding_table.T)
    return out_t.T
